# Initial kernel scaffold; baseline (speedup 1.0000x reference)
import jax
import jax.numpy as jnp
from jax.experimental import pallas as pl


def kernel(x, W_pos, W_word):
    raise NotImplementedError("write your pallas kernel here")


# speedup vs baseline: 1.0371x; 1.0371x over previous
import functools

import jax
import jax.numpy as jnp
from jax import lax
from jax.experimental import pallas as pl
from jax.experimental.pallas import tpu as pltpu
from jax.experimental.pallas import tpu_sc as plsc

NC = 2
NS = 16
NW = NC * NS

BATCH = 4096
SEQ = 200
DIM = 64
BPW = BATCH // NW
LANES = 16
GROUPS = DIM // LANES


def _sc_body(xT_hbm, wpos_hbm, wword_hbm, out_hbm, idx_v, pos_v, rows_v, gsem):
    c = lax.axis_index("c")
    s = lax.axis_index("s")
    wid = s * NC + c
    b0 = wid * BPW

    pltpu.sync_copy(xT_hbm.at[:, pl.ds(b0, BPW)], idx_v)
    pltpu.sync_copy(wpos_hbm.at[pl.ds(0, SEQ)], pos_v)

    def step(l, carry):
        pltpu.async_copy(wword_hbm.at[idx_v.at[l]], rows_v, gsem).wait()
        p = [pos_v[l, pl.ds(LANES * g, LANES)] for g in range(GROUPS)]

        def add_row(r, carry2):
            for g in range(GROUPS):
                sl = pl.ds(LANES * g, LANES)
                rows_v[r, sl] = rows_v[r, sl] + p[g]
            return carry2

        lax.fori_loop(0, BPW, add_row, 0)
        pltpu.sync_copy(rows_v, out_hbm.at[pl.ds(b0, BPW), l])
        return carry

    lax.fori_loop(0, SEQ, step, 0)


@functools.partial(jax.jit, donate_argnums=())
def kernel(x, W_pos, W_word):
    xT = x.T
    mesh = plsc.VectorSubcoreMesh(core_axis_name="c", subcore_axis_name="s",
                                  num_cores=NC, num_subcores=NS)
    run = pl.kernel(
        _sc_body,
        out_type=jax.ShapeDtypeStruct((BATCH, SEQ, DIM), jnp.float32),
        mesh=mesh,
        scratch_types=[
            pltpu.VMEM((SEQ, BPW), jnp.int32),
            pltpu.VMEM((SEQ, DIM), jnp.float32),
            pltpu.VMEM((BPW, DIM), jnp.float32),
            pltpu.SemaphoreType.DMA,
        ],
        compiler_params=pltpu.CompilerParams(use_tc_tiling_on_sc=False),
    )
    return run(xT, W_pos, W_word)


# speedup vs baseline: 1.1787x; 1.1366x over previous
import functools

import jax
import jax.numpy as jnp
from jax import lax
from jax.experimental import pallas as pl
from jax.experimental.pallas import tpu as pltpu
from jax.experimental.pallas import tpu_sc as plsc

NC = 2
NS = 16
NW = NC * NS

BATCH = 4096
SEQ = 200
DIM = 64
BPW = BATCH // NW
LANES = 16
GROUPS = DIM // LANES
RUNROLL = 4


def _sc_body(xT_hbm, wpos_hbm, wword_hbm, out_hbm,
             idx_v, pos_v, gb0, gb1, ob0, ob1,
             gsem0, gsem1, wsem0, wsem1):
    c = lax.axis_index("c")
    s = lax.axis_index("s")
    wid = s * NC + c
    b0 = wid * BPW

    gb = (gb0, gb1)
    ob = (ob0, ob1)
    gsem = (gsem0, gsem1)
    wsem = (wsem0, wsem1)

    pltpu.sync_copy(xT_hbm.at[:, pl.ds(b0, BPW)], idx_v)
    pltpu.sync_copy(wpos_hbm.at[pl.ds(0, SEQ)], pos_v)

    def gather_desc(l, par):
        return pltpu.make_async_copy(wword_hbm.at[idx_v.at[l]], gb[par],
                                     gsem[par])

    def write_desc(l, par):
        return pltpu.make_async_copy(ob[par], out_hbm.at[pl.ds(b0, BPW), l],
                                     wsem[par])

    gather_desc(0, 0).start()
    gather_desc(1, 1).start()

    def step(i, carry):
        for par in range(2):
            l = 2 * i + par
            gather_desc(l, par).wait()
            @pl.when(i >= 1)
            def _():
                write_desc(l - 2, par).wait()
            p = [pos_v[l, pl.ds(LANES * g, LANES)] for g in range(GROUPS)]
            gbuf = gb[par]
            obuf = ob[par]

            def add_block(r4, carry2):
                for rr in range(RUNROLL):
                    r = r4 * RUNROLL + rr
                    for g in range(GROUPS):
                        sl = pl.ds(LANES * g, LANES)
                        obuf[r, sl] = gbuf[r, sl] + p[g]
                return carry2

            lax.fori_loop(0, BPW // RUNROLL, add_block, 0)
            @pl.when(l + 2 < SEQ)
            def _():
                gather_desc(l + 2, par).start()
            write_desc(l, par).start()
        return carry

    lax.fori_loop(0, SEQ // 2, step, 0)
    write_desc(SEQ - 2, 0).wait()
    write_desc(SEQ - 1, 1).wait()


@functools.partial(jax.jit, donate_argnums=())
def kernel(x, W_pos, W_word):
    xT = x.T
    mesh = plsc.VectorSubcoreMesh(core_axis_name="c", subcore_axis_name="s",
                                  num_cores=NC, num_subcores=NS)
    run = pl.kernel(
        _sc_body,
        out_type=jax.ShapeDtypeStruct((BATCH, SEQ, DIM), jnp.float32),
        mesh=mesh,
        scratch_types=[
            pltpu.VMEM((SEQ, BPW), jnp.int32),
            pltpu.VMEM((SEQ, DIM), jnp.float32),
            pltpu.VMEM((BPW, DIM), jnp.float32),
            pltpu.VMEM((BPW, DIM), jnp.float32),
            pltpu.VMEM((BPW, DIM), jnp.float32),
            pltpu.VMEM((BPW, DIM), jnp.float32),
            pltpu.SemaphoreType.DMA,
            pltpu.SemaphoreType.DMA,
            pltpu.SemaphoreType.DMA,
            pltpu.SemaphoreType.DMA,
        ],
        compiler_params=pltpu.CompilerParams(use_tc_tiling_on_sc=False),
    )
    return run(xT, W_pos, W_word)


# speedup vs baseline: 1.8336x; 1.5556x over previous
import functools

import jax
import jax.numpy as jnp
from jax import lax
from jax.experimental import pallas as pl
from jax.experimental.pallas import tpu as pltpu
from jax.experimental.pallas import tpu_sc as plsc

NC = 2
NS = 16
NW = NC * NS

BATCH = 4096
SEQ = 200
DIM = 64
BPW = BATCH // NW
LANES = 16
GROUPS = DIM // LANES
TPAD = BPW + 1


def _sc_body(xT_hbm, pos_hbm, wword_hbm, out_hbm,
             idx_v, pos_v, gb0, gb1, tb0, tb1,
             gsem0, gsem1, wsem0, wsem1):
    c = lax.axis_index("c")
    s = lax.axis_index("s")
    wid = s * NC + c
    b0 = wid * BPW

    gb = (gb0, gb1)
    tb = (tb0, tb1)
    gsem = (gsem0, gsem1)
    wsem = (wsem0, wsem1)

    pltpu.sync_copy(xT_hbm.at[:, pl.ds(b0, BPW)], idx_v)
    pltpu.sync_copy(pos_hbm, pos_v)

    def gather_desc(l, par):
        return pltpu.make_async_copy(wword_hbm.at[idx_v.at[l]], gb[par],
                                     gsem[par])

    def write_desc(l, par):
        return pltpu.make_async_copy(tb[par],
                                     out_hbm.at[pl.ds(b0, BPW), pl.ds(DIM * l, DIM)],
                                     wsem[par])

    gather_desc(0, 0).start()
    gather_desc(1, 1).start()

    def step(i, carry):
        for par in range(2):
            l = 2 * i + par
            gather_desc(l, par).wait()
            @pl.when(i >= 1)
            def _():
                write_desc(l - 2, par).wait()
            p = [pos_v[pl.ds(DIM * l + LANES * g, LANES)] for g in range(GROUPS)]
            gbuf = gb[par]
            tbuf = tb[par]

            def add_row(b, carry2):
                for g in range(GROUPS):
                    sl = pl.ds(LANES * g, LANES)
                    tbuf[b, sl] = gbuf[b, sl] + p[g]
                return carry2

            lax.fori_loop(0, BPW, add_row, 0)
            @pl.when(l + 2 < SEQ)
            def _():
                gather_desc(l + 2, par).start()
            write_desc(l, par).start()
        return carry

    lax.fori_loop(0, SEQ // 2, step, 0)
    write_desc(SEQ - 2, 0).wait()
    write_desc(SEQ - 1, 1).wait()


@functools.partial(jax.jit, donate_argnums=())
def kernel(x, W_pos, W_word):
    xT = x.T
    pos = W_pos[:SEQ].reshape(-1)
    mesh = plsc.VectorSubcoreMesh(core_axis_name="c", subcore_axis_name="s",
                                  num_cores=NC, num_subcores=NS)
    run = pl.kernel(
        _sc_body,
        out_type=jax.ShapeDtypeStruct((BATCH, SEQ * DIM), jnp.float32),
        mesh=mesh,
        scratch_types=[
            pltpu.VMEM((SEQ, BPW), jnp.int32),
            pltpu.VMEM((SEQ * DIM,), jnp.float32),
            pltpu.VMEM((BPW, DIM), jnp.float32),
            pltpu.VMEM((BPW, DIM), jnp.float32),
            pltpu.VMEM((BPW, DIM), jnp.float32),
            pltpu.VMEM((BPW, DIM), jnp.float32),
            pltpu.SemaphoreType.DMA,
            pltpu.SemaphoreType.DMA,
            pltpu.SemaphoreType.DMA,
            pltpu.SemaphoreType.DMA,
        ],
        compiler_params=pltpu.CompilerParams(use_tc_tiling_on_sc=False),
    )
    out2 = run(xT, pos, W_word)
    return out2.reshape(BATCH, SEQ, DIM)
